# baseline (speedup 1.0000x reference)
import functools

import jax
import jax.numpy as jnp
from jax import lax
from jax.experimental import pallas as pl
from jax.experimental.pallas import tpu as pltpu
from jax.experimental.pallas import tpu_sc as plsc

H = 128
W = 128
N = H * W
NW = 32
CHUNK = N // NW
L = 16


def _ifloor(x):
    t = x.astype(jnp.int32)
    return t - (t.astype(jnp.float32) > x).astype(jnp.int32)


_mesh = plsc.VectorSubcoreMesh(core_axis_name="c", subcore_axis_name="s")


@functools.partial(
    pl.kernel,
    mesh=_mesh,
    compiler_params=pltpu.CompilerParams(needs_layout_passes=False),
    out_type=jax.ShapeDtypeStruct((N,), jnp.float32),
    scratch_types=[
        pltpu.VMEM((N,), jnp.float32),
        pltpu.VMEM((CHUNK,), jnp.float32),
        pltpu.VMEM((CHUNK,), jnp.float32),
        pltpu.VMEM((CHUNK,), jnp.float32),
        pltpu.VMEM((4, CHUNK), jnp.int32),
        pltpu.VMEM((4, CHUNK), jnp.float32),
        pltpu.VMEM_SHARED((N,), jnp.float32),
        pltpu.SemaphoreType.DMA,
        pltpu.SemaphoreType.DMA,
    ],
)
def _warp(img_hbm, disp_hbm, out_hbm, img_v, dh_v, dw_v, out_v, idx_v, wgt_v,
          img_s, sem, dsem):
    sid = lax.axis_index("s")
    wid = sid * 2 + lax.axis_index("c")
    base = wid * CHUNK

    odd = sid % 2 == 1

    @pl.when(sid == 0)
    def _():
        pltpu.make_async_copy(img_hbm, img_s, sem).start()

    @pl.when(odd)
    def _():
        pltpu.make_async_copy(img_hbm, img_v, sem).start()

    dh_cp = pltpu.async_copy(disp_hbm.at[pl.ds(base, CHUNK)], dh_v, dsem)
    dw_cp = pltpu.async_copy(disp_hbm.at[pl.ds(N + base, CHUNK)], dw_v, dsem)

    @pl.when(sid == 0)
    def _():
        pltpu.make_async_copy(img_hbm, img_s, sem).wait()

    plsc.subcore_barrier()

    @pl.when(jnp.logical_not(odd))
    def _():
        pltpu.make_async_copy(img_s, img_v, sem).start()

    dh_cp.wait()
    dw_cp.wait()

    lane = lax.broadcasted_iota(jnp.int32, (L,), 0)
    row0f = jnp.broadcast_to(wid * (CHUNK // W), (L,)).astype(jnp.float32)
    zero = jnp.float32(0.0)
    one = jnp.float32(1.0)

    @plsc.parallel_loop(0, CHUNK, step=L, unroll=1)
    def prep(off):
        row_off = jnp.broadcast_to(off // W + 1, (L,))
        hu = dh_v[pl.ds(off, L)] + row0f + row_off.astype(jnp.float32)
        wu = dw_v[pl.ds(off, L)] + (lane + (off % W + 1)).astype(jnp.float32)
        hf_u = _ifloor(hu)
        wf_u = _ifloor(wu)
        hm = hf_u - 1
        wm = wf_u - 1
        ihf = jnp.clip(hm, 0, H - 1) * W
        ihc = jnp.clip(hf_u, 0, H - 1) * W
        iwf = jnp.clip(wm, 0, W - 1)
        iwc = jnp.clip(wf_u, 0, W - 1)
        vhf = hm.astype(jnp.uint32) < jnp.uint32(H)
        vhc = hf_u.astype(jnp.uint32) < jnp.uint32(H)
        vwf = wm.astype(jnp.uint32) < jnp.uint32(W)
        vwc = wf_u.astype(jnp.uint32) < jnp.uint32(W)
        d_h = jnp.clip(hf_u + 1, 0, H + 1).astype(jnp.float32) - hu
        d_w = jnp.clip(wf_u + 1, 0, W + 1).astype(jnp.float32) - wu
        ah0 = jnp.where(vhf, d_h, zero)
        ah1 = jnp.where(vhc, one - d_h, zero)
        aw0 = jnp.where(vwf, d_w, zero)
        aw1 = jnp.where(vwc, one - d_w, zero)
        idx_v[0, pl.ds(off, L)] = ihf + iwf
        idx_v[1, pl.ds(off, L)] = ihc + iwf
        idx_v[2, pl.ds(off, L)] = ihf + iwc
        idx_v[3, pl.ds(off, L)] = ihc + iwc
        wgt_v[0, pl.ds(off, L)] = aw0 * ah0
        wgt_v[1, pl.ds(off, L)] = aw0 * ah1
        wgt_v[2, pl.ds(off, L)] = aw1 * ah0
        wgt_v[3, pl.ds(off, L)] = aw1 * ah1

    pltpu.make_async_copy(img_hbm, img_v, sem).wait()

    @plsc.parallel_loop(0, CHUNK, step=L, unroll=1)
    def blend(off):
        acc = (plsc.load_gather(img_v, [idx_v[0, pl.ds(off, L)]])
               * wgt_v[0, pl.ds(off, L)])
        acc += (plsc.load_gather(img_v, [idx_v[1, pl.ds(off, L)]])
                * wgt_v[1, pl.ds(off, L)])
        acc += (plsc.load_gather(img_v, [idx_v[2, pl.ds(off, L)]])
                * wgt_v[2, pl.ds(off, L)])
        acc += (plsc.load_gather(img_v, [idx_v[3, pl.ds(off, L)]])
                * wgt_v[3, pl.ds(off, L)])
        out_v[pl.ds(off, L)] = acc

    pltpu.sync_copy(out_v, out_hbm.at[pl.ds(base, CHUNK)])


def kernel(input1, input2):
    img = input1.reshape(N)
    disp = input2.reshape(2 * N)
    out = _warp(img, disp)
    return out.reshape(1, 1, H, W)


# speedup vs baseline: 1.0222x; 1.0222x over previous
import functools

import jax
import jax.numpy as jnp
from jax import lax
from jax.experimental import pallas as pl
from jax.experimental.pallas import tpu as pltpu
from jax.experimental.pallas import tpu_sc as plsc

H = 128
W = 128
N = H * W
NW = 32
CHUNK = N // NW
L = 16


def _ifloor(x):
    t = x.astype(jnp.int32)
    return t - (t.astype(jnp.float32) > x).astype(jnp.int32)


_mesh = plsc.VectorSubcoreMesh(core_axis_name="c", subcore_axis_name="s")


@functools.partial(
    pl.kernel,
    mesh=_mesh,
    compiler_params=pltpu.CompilerParams(needs_layout_passes=False),
    out_type=jax.ShapeDtypeStruct((N,), jnp.float32),
    scratch_types=[
        pltpu.VMEM((N,), jnp.float32),
        pltpu.VMEM((CHUNK,), jnp.float32),
        pltpu.VMEM((CHUNK,), jnp.float32),
        pltpu.VMEM((CHUNK,), jnp.float32),
        pltpu.VMEM((4, CHUNK), jnp.int32),
        pltpu.VMEM((4, CHUNK), jnp.float32),
        pltpu.VMEM_SHARED((N,), jnp.float32),
        pltpu.SemaphoreType.DMA,
        pltpu.SemaphoreType.DMA,
    ],
)
def _warp(img_hbm, disp_hbm, out_hbm, img_v, dh_v, dw_v, out_v, idx_v, wgt_v,
          img_s, sem, dsem):
    sid = lax.axis_index("s")
    wid = sid * 2 + lax.axis_index("c")
    base = wid * CHUNK

    @pl.when(sid == 0)
    def _():
        pltpu.make_async_copy(img_hbm, img_s, sem).start()

    dh_cp = pltpu.async_copy(disp_hbm.at[pl.ds(base, CHUNK)], dh_v, dsem)
    dw_cp = pltpu.async_copy(disp_hbm.at[pl.ds(N + base, CHUNK)], dw_v, dsem)

    @pl.when(sid == 0)
    def _():
        pltpu.make_async_copy(img_hbm, img_s, sem).wait()

    plsc.subcore_barrier()
    fan_cp = pltpu.async_copy(img_s, img_v, sem)
    dh_cp.wait()
    dw_cp.wait()

    lane = lax.broadcasted_iota(jnp.int32, (L,), 0)
    row0f = jnp.broadcast_to(wid * (CHUNK // W), (L,)).astype(jnp.float32)
    zero = jnp.float32(0.0)
    one = jnp.float32(1.0)

    @plsc.parallel_loop(0, CHUNK, step=L, unroll=1)
    def prep(off):
        row_off = jnp.broadcast_to(off // W + 1, (L,))
        hu = dh_v[pl.ds(off, L)] + row0f + row_off.astype(jnp.float32)
        wu = dw_v[pl.ds(off, L)] + (lane + (off % W + 1)).astype(jnp.float32)
        hf_u = _ifloor(hu)
        wf_u = _ifloor(wu)
        hm = hf_u - 1
        wm = wf_u - 1
        ihf = jnp.clip(hm, 0, H - 1) * W
        ihc = jnp.clip(hf_u, 0, H - 1) * W
        iwf = jnp.clip(wm, 0, W - 1)
        iwc = jnp.clip(wf_u, 0, W - 1)
        vhf = hm.astype(jnp.uint32) < jnp.uint32(H)
        vhc = hf_u.astype(jnp.uint32) < jnp.uint32(H)
        vwf = wm.astype(jnp.uint32) < jnp.uint32(W)
        vwc = wf_u.astype(jnp.uint32) < jnp.uint32(W)
        d_h = jnp.clip(hf_u + 1, 0, H + 1).astype(jnp.float32) - hu
        d_w = jnp.clip(wf_u + 1, 0, W + 1).astype(jnp.float32) - wu
        ah0 = jnp.where(vhf, d_h, zero)
        ah1 = jnp.where(vhc, one - d_h, zero)
        aw0 = jnp.where(vwf, d_w, zero)
        aw1 = jnp.where(vwc, one - d_w, zero)
        idx_v[0, pl.ds(off, L)] = ihf + iwf
        idx_v[1, pl.ds(off, L)] = ihc + iwf
        idx_v[2, pl.ds(off, L)] = ihf + iwc
        idx_v[3, pl.ds(off, L)] = ihc + iwc
        wgt_v[0, pl.ds(off, L)] = aw0 * ah0
        wgt_v[1, pl.ds(off, L)] = aw0 * ah1
        wgt_v[2, pl.ds(off, L)] = aw1 * ah0
        wgt_v[3, pl.ds(off, L)] = aw1 * ah1

    fan_cp.wait()

    @plsc.parallel_loop(0, CHUNK, step=L, unroll=1)
    def blend(off):
        acc = (plsc.load_gather(img_v, [idx_v[0, pl.ds(off, L)]])
               * wgt_v[0, pl.ds(off, L)])
        acc += (plsc.load_gather(img_v, [idx_v[1, pl.ds(off, L)]])
                * wgt_v[1, pl.ds(off, L)])
        acc += (plsc.load_gather(img_v, [idx_v[2, pl.ds(off, L)]])
                * wgt_v[2, pl.ds(off, L)])
        acc += (plsc.load_gather(img_v, [idx_v[3, pl.ds(off, L)]])
                * wgt_v[3, pl.ds(off, L)])
        out_v[pl.ds(off, L)] = acc

    pltpu.sync_copy(out_v, out_hbm.at[pl.ds(base, CHUNK)])


def kernel(input1, input2):
    img = input1.reshape(N)
    disp = input2.reshape(2 * N)
    out = _warp(img, disp)
    return out.reshape(1, 1, H, W)
